# baseline (speedup 1.0000x reference)
import functools

import jax
import jax.numpy as jnp
from jax import lax
from jax.experimental import pallas as pl
from jax.experimental.pallas import tpu as pltpu


def _extrema_kernel(x_hbm, o_hbm, xbuf, zbuf, acc_ref, zsems, ssems, rsem, *,
                    n: int, c_dim: int, pool_size: int, width: int,
                    zrows: int, n_chunks: int):
    pw = xbuf.shape[1]
    row = o_hbm.shape[1]

    pltpu.make_async_copy(x_hbm.at[:, pl.ds(0, pw)], xbuf, rsem).start()

    zbuf[...] = jnp.zeros(zbuf.shape, zbuf.dtype)
    for k in range(n_chunks):
        pltpu.make_async_copy(
            zbuf, o_hbm.at[pl.ds(k * zrows, zrows), :], zsems.at[k]).start()

    pltpu.make_async_copy(x_hbm.at[:, pl.ds(0, pw)], xbuf, rsem).wait()
    xw = xbuf[...].reshape(n, c_dim, pw)
    lane = lax.broadcasted_iota(jnp.int32, xw.shape, 2)
    inwin = lane % width < pool_size
    aw = jnp.where(inwin, jnp.abs(xw), -1.0)
    m = jnp.max(aw, axis=-1, keepdims=True)
    cand = jnp.where(aw == m, lane, jnp.int32(pw))
    idx = jnp.min(cand, axis=-1, keepdims=True)
    col = lax.broadcasted_iota(jnp.int32, (1, pw), 1)
    hit = col == idx[:, 0, :]
    for c in range(1, c_dim):
        hit = hit | (col == idx[:, c, :])
    acc_ref[...] = jnp.where(hit, xw[:, 0, :], 0.0).astype(acc_ref.dtype)

    for k in range(n_chunks):
        pltpu.make_async_copy(
            zbuf, o_hbm.at[pl.ds(k * zrows, zrows), :], zsems.at[k]).wait()
        pltpu.make_async_copy(
            acc_ref.at[pl.ds(k * zrows, zrows), :],
            o_hbm.at[pl.ds(k * zrows, zrows), pl.ds(0, pw)],
            ssems.at[k],
        ).start()
    for k in range(n_chunks):
        pltpu.make_async_copy(
            acc_ref.at[pl.ds(k * zrows, zrows), :],
            o_hbm.at[pl.ds(k * zrows, zrows), pl.ds(0, pw)],
            ssems.at[k],
        ).wait()


def _extrema_pool_indices_2d(x, pool_size: int):
    N, C, H, W = x.shape
    HW = H * W
    row = C * HW
    x2 = x.reshape(N * C, HW)
    pw = pool_size * W

    zrows = min(256, N)
    n_chunks = N // zrows

    out2 = pl.pallas_call(
        functools.partial(_extrema_kernel, n=N, c_dim=C, pool_size=pool_size,
                          width=W, zrows=zrows, n_chunks=n_chunks),
        out_shape=jax.ShapeDtypeStruct((N, row), x.dtype),
        in_specs=[pl.BlockSpec(memory_space=pl.ANY)],
        out_specs=pl.BlockSpec(memory_space=pl.ANY),
        scratch_shapes=[
            pltpu.VMEM((N * C, pw), x.dtype),
            pltpu.VMEM((zrows, row), x.dtype),
            pltpu.VMEM((N, pw), x.dtype),
            pltpu.SemaphoreType.DMA((n_chunks,)),
            pltpu.SemaphoreType.DMA((n_chunks,)),
            pltpu.SemaphoreType.DMA,
        ],
        compiler_params=pltpu.CompilerParams(
            vmem_limit_bytes=64 * 1024 * 1024,
        ),
        cost_estimate=pl.CostEstimate(
            flops=10 * N * C * pw,
            transcendentals=0,
            bytes_accessed=(N * row + N * C * pw) * x.dtype.itemsize,
        ),
    )(x2)
    return out2.reshape(N, C, H, W)


def kernel(x):
    return _extrema_pool_indices_2d(x, 4)


# speedup vs baseline: 3.5741x; 3.5741x over previous
import functools

import jax
import jax.numpy as jnp
from jax import lax
from jax.experimental import pallas as pl
from jax.experimental.pallas import tpu as pltpu

_LANE = 128


def _extrema_kernel(win_ref, o_hbm, bufs, sems, *, pool_size: int,
                    width: int, region: int, zrows: int, n_chunks: int):
    n, c_dim, pp = win_ref.shape
    row = o_hbm.shape[1]

    bufs[...] = jnp.zeros(bufs.shape, bufs.dtype)

    wcol = lax.broadcasted_iota(jnp.int32, (1, pp), 1)
    dcol = lax.broadcasted_iota(jnp.int32, (1, region), 1)

    for k in range(n_chunks):
        slot = k % 2
        if k >= 2:
            pltpu.make_async_copy(
                bufs.at[slot], o_hbm.at[pl.ds((k - 2) * zrows, zrows), :],
                sems.at[slot]).wait()
        win = win_ref[pl.ds(k * zrows, zrows)]
        awin = jnp.abs(win)
        m = jnp.max(awin, axis=-1, keepdims=True)
        cand = jnp.where(awin == m, wcol, jnp.int32(pp))
        jidx = jnp.min(cand, axis=-1, keepdims=True)
        hit = wcol == jidx[:, 0, :]
        for c in range(1, c_dim):
            hit = hit | (wcol == jidx[:, c, :])
        strip = jnp.where(hit, win[:, 0, :], 0.0)
        acc = jnp.zeros((zrows, region), bufs.dtype)
        for j in range(pp):
            acc = jnp.where(dcol == (j // pool_size) * width + j % pool_size,
                            strip[:, j:j + 1], acc)
        bufs[slot, :, :region] = acc
        pltpu.make_async_copy(
            bufs.at[slot], o_hbm.at[pl.ds(k * zrows, zrows), :],
            sems.at[slot]).start()

    for k in range(max(0, n_chunks - 2), n_chunks):
        pltpu.make_async_copy(
            bufs.at[k % 2], o_hbm.at[pl.ds(k * zrows, zrows), :],
            sems.at[k % 2]).wait()


def _extrema_pool_indices_2d(x, pool_size: int):
    N, C, H, W = x.shape
    HW = H * W
    pp = pool_size * pool_size
    row = C * HW
    win = x[:, :, :pool_size, :pool_size].reshape(N, C, pp)

    region = min(-(-(pool_size * W) // _LANE) * _LANE, row)
    zrows = min(256, N)
    n_chunks = N // zrows

    out2 = pl.pallas_call(
        functools.partial(_extrema_kernel, pool_size=pool_size, width=W,
                          region=region, zrows=zrows, n_chunks=n_chunks),
        out_shape=jax.ShapeDtypeStruct((N, row), x.dtype),
        in_specs=[pl.BlockSpec((N, C, pp), lambda: (0, 0, 0))],
        out_specs=pl.BlockSpec(memory_space=pl.ANY),
        scratch_shapes=[
            pltpu.VMEM((2, zrows, row), x.dtype),
            pltpu.SemaphoreType.DMA((2,)),
        ],
        compiler_params=pltpu.CompilerParams(
            vmem_limit_bytes=64 * 1024 * 1024,
        ),
        cost_estimate=pl.CostEstimate(
            flops=10 * N * C * pp + 2 * N * region,
            transcendentals=0,
            bytes_accessed=(N * row + N * C * pp) * x.dtype.itemsize,
        ),
    )(win)
    return out2.reshape(N, C, H, W)


def kernel(x):
    return _extrema_pool_indices_2d(x, 4)
